# baseline (speedup 1.0000x reference)
import functools

import jax
import jax.numpy as jnp
from jax import lax
from jax.experimental import pallas as pl
from jax.experimental.pallas import tpu as pltpu
from jax.experimental.pallas import tpu_sc as plsc

N = 10000
D = 128
NP = 10240
E = 320000
NW = 32
CH = 128
NCH = 80
HCH = 40
PER_W = CH * NCH
E_PAD = NW * PER_W
RPT = NP // 16
BLK = 1024

def _deg_body(dst_hbm, out_hbm, dst_v, ones_v, zeros_v, acc_sh):
    c = lax.axis_index("c")
    s = lax.axis_index("s")
    wid = c * 16 + s
    for k in range(CH // 16):
        ones_v[pl.ds(k * 16, 16)] = jnp.ones((16,), jnp.float32)
    for k in range(RPT // 16):
        zeros_v[pl.ds(k * 16, 16)] = jnp.zeros((16,), jnp.float32)
    pltpu.sync_copy(zeros_v, acc_sh.at[pl.ds(s * RPT, RPT)])
    plsc.subcore_barrier()

    def body(j, carry):
        off = wid * PER_W + j * CH
        pltpu.sync_copy(dst_hbm.at[pl.ds(off, CH)], dst_v)
        pltpu.sync_copy(ones_v, acc_sh.at[dst_v], add=True)
        return carry

    lax.fori_loop(0, NCH, body, 0)
    plsc.subcore_barrier()
    pltpu.sync_copy(acc_sh.at[pl.ds(s * RPT, RPT)],
                    out_hbm.at[c, pl.ds(s * RPT, RPT)])


def _edge_body(hs_hbm, src_hbm, dst_hbm, out_hbm,
               src_v, dst_v, rows_v, zero_v, acc_sh, sem):
    c = lax.axis_index("c")
    s = lax.axis_index("s")
    wid = c * 16 + s
    z16 = jnp.zeros((16,), jnp.float32)
    for r in range(16):
        for k in range(D // 16):
            zero_v[r, pl.ds(k * 16, 16)] = z16

    def zbody(i, carry):
        pltpu.sync_copy(zero_v, acc_sh.at[pl.ds(s * RPT + i * 16, 16)])
        return carry

    lax.fori_loop(0, RPT // 16, zbody, 0)
    plsc.subcore_barrier()

    def body(j, carry):
        off = wid * PER_W + j * CH
        pltpu.sync_copy(src_hbm.at[pl.ds(off, CH)], src_v)
        pltpu.sync_copy(dst_hbm.at[pl.ds(off, CH)], dst_v)
        pltpu.async_copy(hs_hbm.at[src_v], rows_v, sem).wait()
        pltpu.sync_copy(rows_v, acc_sh.at[dst_v], add=True)
        return carry

    lax.fori_loop(0, NCH, body, 0)
    plsc.subcore_barrier()
    pltpu.sync_copy(acc_sh.at[pl.ds(s * RPT, RPT)],
                    out_hbm.at[c, pl.ds(s * RPT, RPT)])


@functools.cache
def _sc_kernels():
    mesh = plsc.VectorSubcoreMesh(core_axis_name="c", subcore_axis_name="s",
                                  num_cores=2, num_subcores=16)
    deg = pl.kernel(
        _deg_body,
        out_type=jax.ShapeDtypeStruct((2, NP), jnp.float32),
        mesh=mesh,
        scratch_types=[
            pltpu.VMEM((CH,), jnp.int32),
            pltpu.VMEM((CH,), jnp.float32),
            pltpu.VMEM((RPT,), jnp.float32),
            pltpu.VMEM_SHARED((NP,), jnp.float32),
        ],
    )
    edge = pl.kernel(
        _edge_body,
        out_type=jax.ShapeDtypeStruct((2, NP, D), jnp.float32),
        mesh=mesh,
        scratch_types=[
            pltpu.VMEM((CH,), jnp.int32),
            pltpu.VMEM((CH,), jnp.int32),
            pltpu.VMEM((CH, D), jnp.float32),
            pltpu.VMEM((16, D), jnp.float32),
            pltpu.VMEM_SHARED((NP, D), jnp.float32),
            pltpu.SemaphoreType.DMA,
        ],
    )
    return deg, edge


def _dis_block(cnt_ref, pid):
    cnt = cnt_ref[0] + cnt_ref[1]
    row = lax.broadcasted_iota(jnp.int32, (BLK, 1), 0) + pid * BLK
    return jnp.where(row < N, lax.rsqrt(cnt + 1.0), 0.0)


def _head_body(cnt_ref, x_ref, w_ref, out_ref):
    dis = _dis_block(cnt_ref, pl.program_id(0))
    out_ref[...] = dis * jnp.dot(x_ref[...], w_ref[...],
                                 preferred_element_type=jnp.float32)


def _layer_body(cnt_ref, acc_ref, hs_ref, b_ref, w_ref, out_ref):
    dis = _dis_block(cnt_ref, pl.program_id(0))
    t = dis * (acc_ref[0] + acc_ref[1] + hs_ref[...]) + b_ref[...]
    a = jnp.maximum(t, 0.0)
    out_ref[...] = dis * jnp.dot(a, w_ref[...],
                                 preferred_element_type=jnp.float32)


def _tail_body(cnt_ref, acc_ref, hs_ref, b_ref, out_ref):
    dis = _dis_block(cnt_ref, pl.program_id(0))
    out_ref[...] = dis * (acc_ref[0] + acc_ref[1] + hs_ref[...]) + b_ref[...]


_CNT_SPEC = pl.BlockSpec((2, BLK, 1), lambda i: (0, i, 0))
_ROW_SPEC = pl.BlockSpec((BLK, D), lambda i: (i, 0))
_ACC_SPEC = pl.BlockSpec((2, BLK, D), lambda i: (0, i, 0))
_W_SPEC = pl.BlockSpec((D, D), lambda i: (0, 0))
_B_SPEC = pl.BlockSpec((1, D), lambda i: (0, 0))
_GRID = NP // BLK
_OUT_SHAPE = jax.ShapeDtypeStruct((NP, D), jnp.float32)

_head = pl.pallas_call(
    _head_body, grid=(_GRID,),
    in_specs=[_CNT_SPEC, _ROW_SPEC, _W_SPEC],
    out_specs=_ROW_SPEC, out_shape=_OUT_SHAPE)

_layer = pl.pallas_call(
    _layer_body, grid=(_GRID,),
    in_specs=[_CNT_SPEC, _ACC_SPEC, _ROW_SPEC, _B_SPEC, _W_SPEC],
    out_specs=_ROW_SPEC, out_shape=_OUT_SHAPE)

_tail = pl.pallas_call(
    _tail_body, grid=(_GRID,),
    in_specs=[_CNT_SPEC, _ACC_SPEC, _ROW_SPEC, _B_SPEC],
    out_specs=_ROW_SPEC, out_shape=_OUT_SHAPE)


def kernel(x, edge_index, W1, b1, W2, b2, W3, b3):
    src = edge_index[0].astype(jnp.int32)
    dst = edge_index[1].astype(jnp.int32)
    pad = jnp.full((E_PAD - E,), N, jnp.int32)
    src_p = jnp.concatenate([src, pad])
    dst_p = jnp.concatenate([dst, pad])
    xp = jnp.pad(x, ((0, NP - N), (0, 0)))

    _deg, _edge_pass = _sc_kernels()
    cnt = _deg(dst_p)[:, :, None]

    hs1 = _head(cnt, xp, W1)
    acc1 = _edge_pass(hs1, src_p, dst_p)
    hs2 = _layer(cnt, acc1, hs1, b1[None, :], W2)
    acc2 = _edge_pass(hs2, src_p, dst_p)
    hs3 = _layer(cnt, acc2, hs2, b2[None, :], W3)
    acc3 = _edge_pass(hs3, src_p, dst_p)
    out = _tail(cnt, acc3, hs3, b3[None, :])
    return out[:N]


# speedup vs baseline: 1.5933x; 1.5933x over previous
import functools

import jax
import jax.numpy as jnp
from jax import lax
from jax.experimental import pallas as pl
from jax.experimental.pallas import tpu as pltpu
from jax.experimental.pallas import tpu_sc as plsc

N = 10000
D = 128
NP = 10240
E = 320000
NW = 32
CH = 128
NCH = 79
PER_W = CH * NCH
E_PAD = NW * PER_W
RPT = NP // 16
BLK = 1024

def _deg_body(dst_hbm, out_hbm, dst_v, ones_v, zeros_v, acc_sh):
    c = lax.axis_index("c")
    s = lax.axis_index("s")
    wid = c * 16 + s
    for k in range(CH // 16):
        ones_v[pl.ds(k * 16, 16)] = jnp.ones((16,), jnp.float32)
    for k in range(RPT // 16):
        zeros_v[pl.ds(k * 16, 16)] = jnp.zeros((16,), jnp.float32)
    pltpu.sync_copy(zeros_v, acc_sh.at[pl.ds(s * RPT, RPT)])
    plsc.subcore_barrier()

    def body(j, carry):
        off = wid * PER_W + j * CH
        pltpu.sync_copy(dst_hbm.at[pl.ds(off, CH)], dst_v)
        pltpu.sync_copy(ones_v, acc_sh.at[dst_v], add=True)
        return carry

    lax.fori_loop(0, NCH, body, 0)
    plsc.subcore_barrier()
    pltpu.sync_copy(acc_sh.at[pl.ds(s * RPT, RPT)],
                    out_hbm.at[c, pl.ds(s * RPT, RPT)])


def _edge_body(hs_hbm, src_hbm, dst_hbm, out_hbm,
               src_v, dst_v, rows_v, zero_v, acc_sh, sem):
    c = lax.axis_index("c")
    s = lax.axis_index("s")
    wid = c * 16 + s
    z16 = jnp.zeros((16,), jnp.float32)
    for r in range(16):
        for k in range(D // 16):
            zero_v[r, pl.ds(k * 16, 16)] = z16

    def zbody(i, carry):
        pltpu.sync_copy(zero_v, acc_sh.at[pl.ds(s * RPT + i * 16, 16)])
        return carry

    lax.fori_loop(0, RPT // 16, zbody, 0)
    plsc.subcore_barrier()

    def body(j, carry):
        off = wid * PER_W + j * CH
        pltpu.sync_copy(src_hbm.at[pl.ds(off, CH)], src_v)
        pltpu.sync_copy(dst_hbm.at[pl.ds(off, CH)], dst_v)
        pltpu.async_copy(hs_hbm.at[src_v], rows_v, sem).wait()
        pltpu.sync_copy(rows_v, acc_sh.at[dst_v], add=True)
        return carry

    lax.fori_loop(0, NCH, body, 0)
    plsc.subcore_barrier()
    pltpu.sync_copy(acc_sh.at[pl.ds(s * RPT, RPT)],
                    out_hbm.at[c, pl.ds(s * RPT, RPT)])


@functools.cache
def _sc_kernels():
    mesh = plsc.VectorSubcoreMesh(core_axis_name="c", subcore_axis_name="s",
                                  num_cores=2, num_subcores=16)
    deg = pl.kernel(
        _deg_body,
        out_type=jax.ShapeDtypeStruct((2, NP), jnp.float32),
        mesh=mesh,
        scratch_types=[
            pltpu.VMEM((CH,), jnp.int32),
            pltpu.VMEM((CH,), jnp.float32),
            pltpu.VMEM((RPT,), jnp.float32),
            pltpu.VMEM_SHARED((NP,), jnp.float32),
        ],
    )
    edge = pl.kernel(
        _edge_body,
        out_type=jax.ShapeDtypeStruct((2, NP, D), jnp.float32),
        mesh=mesh,
        scratch_types=[
            pltpu.VMEM((CH,), jnp.int32),
            pltpu.VMEM((CH,), jnp.int32),
            pltpu.VMEM((CH, D), jnp.float32),
            pltpu.VMEM((16, D), jnp.float32),
            pltpu.VMEM_SHARED((NP, D), jnp.float32),
            pltpu.SemaphoreType.DMA,
        ],
    )
    return deg, edge


def _dis_block(cnt_ref, pid):
    cnt = cnt_ref[0] + cnt_ref[1]
    row = lax.broadcasted_iota(jnp.int32, (BLK, 1), 0) + pid * BLK
    return jnp.where(row < N, lax.rsqrt(cnt + 1.0), 0.0)


def _head_body(cnt_ref, x_ref, w_ref, out_ref):
    dis = _dis_block(cnt_ref, pl.program_id(0))
    out_ref[...] = dis * jnp.dot(x_ref[...], w_ref[...],
                                 preferred_element_type=jnp.float32)


def _layer_body(cnt_ref, acc_ref, hs_ref, b_ref, w_ref, out_ref):
    dis = _dis_block(cnt_ref, pl.program_id(0))
    t = dis * (acc_ref[0] + acc_ref[1] + hs_ref[...]) + b_ref[...]
    a = jnp.maximum(t, 0.0)
    out_ref[...] = dis * jnp.dot(a, w_ref[...],
                                 preferred_element_type=jnp.float32)


def _tail_body(cnt_ref, acc_ref, hs_ref, b_ref, out_ref):
    dis = _dis_block(cnt_ref, pl.program_id(0))
    out_ref[...] = dis * (acc_ref[0] + acc_ref[1] + hs_ref[...]) + b_ref[...]


_CNT_SPEC = pl.BlockSpec((2, BLK, 1), lambda i: (0, i, 0))
_ROW_SPEC = pl.BlockSpec((BLK, D), lambda i: (i, 0))
_ACC_SPEC = pl.BlockSpec((2, BLK, D), lambda i: (0, i, 0))
_W_SPEC = pl.BlockSpec((D, D), lambda i: (0, 0))
_B_SPEC = pl.BlockSpec((1, D), lambda i: (0, 0))
_GRID = NP // BLK
_OUT_SHAPE = jax.ShapeDtypeStruct((NP, D), jnp.float32)

_head = pl.pallas_call(
    _head_body, grid=(_GRID,),
    in_specs=[_CNT_SPEC, _ROW_SPEC, _W_SPEC],
    out_specs=_ROW_SPEC, out_shape=_OUT_SHAPE)

_layer = pl.pallas_call(
    _layer_body, grid=(_GRID,),
    in_specs=[_CNT_SPEC, _ACC_SPEC, _ROW_SPEC, _B_SPEC, _W_SPEC],
    out_specs=_ROW_SPEC, out_shape=_OUT_SHAPE)

_tail = pl.pallas_call(
    _tail_body, grid=(_GRID,),
    in_specs=[_CNT_SPEC, _ACC_SPEC, _ROW_SPEC, _B_SPEC],
    out_specs=_ROW_SPEC, out_shape=_OUT_SHAPE)


def kernel(x, edge_index, W1, b1, W2, b2, W3, b3):
    src = edge_index[0].astype(jnp.int32)
    dst = edge_index[1].astype(jnp.int32)
    pad = jnp.full((E_PAD - E,), N, jnp.int32)
    src_p = jnp.concatenate([src, pad])
    dst_p = jnp.concatenate([dst, pad])
    xp = jnp.pad(x, ((0, NP - N), (0, 0)))

    _deg, _edge_pass = _sc_kernels()
    cnt = _deg(dst_p)[:, :, None]

    hs1 = _head(cnt, xp, W1)
    acc1 = _edge_pass(hs1, src_p, dst_p)
    hs2 = _layer(cnt, acc1, hs1, b1[None, :], W2)
    acc2 = _edge_pass(hs2, src_p, dst_p)
    hs3 = _layer(cnt, acc2, hs2, b2[None, :], W3)
    acc3 = _edge_pass(hs3, src_p, dst_p)
    out = _tail(cnt, acc3, hs3, b3[None, :])
    return out[:N]


# speedup vs baseline: 1.8734x; 1.1758x over previous
import functools

import jax
import jax.numpy as jnp
from jax import lax
from jax.experimental import pallas as pl
from jax.experimental.pallas import tpu as pltpu
from jax.experimental.pallas import tpu_sc as plsc

N = 10000
D = 128
NP = 10240
E = 320000
NW = 32
CH = 128
NCH = 79
HD = D // 2
NCHT = 2 * NCH
PER_W = CH * NCH
E_PAD = NW * PER_W
RPT = NP // 16
BLK = 1024

def _deg_body(dst_hbm, out_hbm, dst_v, ones_v, zeros_v, acc_sh):
    c = lax.axis_index("c")
    s = lax.axis_index("s")
    wid = c * 16 + s
    for k in range(CH // 16):
        ones_v[pl.ds(k * 16, 16)] = jnp.ones((16,), jnp.float32)
    for k in range(RPT // 16):
        zeros_v[pl.ds(k * 16, 16)] = jnp.zeros((16,), jnp.float32)
    pltpu.sync_copy(zeros_v, acc_sh.at[pl.ds(s * RPT, RPT)])
    plsc.subcore_barrier()

    def body(j, carry):
        off = wid * PER_W + j * CH
        pltpu.sync_copy(dst_hbm.at[pl.ds(off, CH)], dst_v)
        pltpu.sync_copy(ones_v, acc_sh.at[dst_v], add=True)
        return carry

    lax.fori_loop(0, NCH, body, 0)
    plsc.subcore_barrier()
    pltpu.sync_copy(acc_sh.at[pl.ds(s * RPT, RPT)],
                    out_hbm.at[c, pl.ds(s * RPT, RPT)])


def _edge_body(hs_hbm, src_hbm, dst_hbm, out_hbm,
               src_a, dst_a, src_b, dst_b, rows_v, zero_v, acc_sh,
               sem, sem_ia, sem_ib):
    c = lax.axis_index("c")
    s = lax.axis_index("s")
    wid = c * 16 + s
    base = wid * PER_W
    z16 = jnp.zeros((16,), jnp.float32)
    for r in range(16):
        for k in range(D // 16):
            zero_v[r, pl.ds(k * 16, 16)] = z16

    def zbody(i, carry):
        pltpu.sync_copy(zero_v, acc_sh.at[pl.ds(s * RPT + i * 16, 16)])
        return carry

    lax.fori_loop(0, RPT // 16, zbody, 0)

    def load_idx(j, sv, dv, sem_i):
        pltpu.async_copy(src_hbm.at[pl.ds(base + j * CH, CH)], sv, sem_i)
        pltpu.async_copy(dst_hbm.at[pl.ds(base + j * CH, CH)], dv, sem_i)

    def wait_idx(sv, dv, sem_i):
        pltpu.make_async_copy(src_hbm.at[pl.ds(base, CH)], sv, sem_i).wait()
        pltpu.make_async_copy(dst_hbm.at[pl.ds(base, CH)], dv, sem_i).wait()

    def do_chunk(sv, dv):
        pltpu.async_copy(hs_hbm.at[sv], rows_v, sem).wait()
        pltpu.sync_copy(rows_v, acc_sh.at[dv], add=True)

    load_idx(0, src_a, dst_a, sem_ia)
    load_idx(1, src_b, dst_b, sem_ib)
    plsc.subcore_barrier()

    def body(k, carry):
        j0 = 2 * k
        wait_idx(src_a, dst_a, sem_ia)
        do_chunk(src_a, dst_a)
        pltpu.async_copy(src_hbm.at[pl.ds(base + (j0 + 2) * CH, CH)],
                         src_a, sem_ia)
        pltpu.async_copy(dst_hbm.at[pl.ds(base + (j0 + 2) * CH, CH)],
                         dst_a, sem_ia)
        wait_idx(src_b, dst_b, sem_ib)
        do_chunk(src_b, dst_b)

        @pl.when(k < (NCH - 1) // 2 - 1)
        def _():
            pltpu.async_copy(src_hbm.at[pl.ds(base + (j0 + 3) * CH, CH)],
                             src_b, sem_ib)
            pltpu.async_copy(dst_hbm.at[pl.ds(base + (j0 + 3) * CH, CH)],
                             dst_b, sem_ib)
        return carry

    lax.fori_loop(0, (NCH - 1) // 2, body, 0)
    wait_idx(src_a, dst_a, sem_ia)
    do_chunk(src_a, dst_a)
    plsc.subcore_barrier()
    pltpu.sync_copy(acc_sh.at[pl.ds(s * RPT, RPT)],
                    out_hbm.at[c, pl.ds(s * RPT, RPT)])


@functools.cache
def _sc_kernels():
    mesh = plsc.VectorSubcoreMesh(core_axis_name="c", subcore_axis_name="s",
                                  num_cores=2, num_subcores=16)
    deg = pl.kernel(
        _deg_body,
        out_type=jax.ShapeDtypeStruct((2, NP), jnp.float32),
        mesh=mesh,
        scratch_types=[
            pltpu.VMEM((CH,), jnp.int32),
            pltpu.VMEM((CH,), jnp.float32),
            pltpu.VMEM((RPT,), jnp.float32),
            pltpu.VMEM_SHARED((NP,), jnp.float32),
        ],
    )
    edge = pl.kernel(
        _edge_body,
        out_type=jax.ShapeDtypeStruct((2, NP, D), jnp.float32),
        mesh=mesh,
        scratch_types=[
            pltpu.VMEM((CH,), jnp.int32),
            pltpu.VMEM((CH,), jnp.int32),
            pltpu.VMEM((CH,), jnp.int32),
            pltpu.VMEM((CH,), jnp.int32),
            pltpu.VMEM((CH, D), jnp.float32),
            pltpu.VMEM((16, D), jnp.float32),
            pltpu.VMEM_SHARED((NP, D), jnp.float32),
            pltpu.SemaphoreType.DMA,
            pltpu.SemaphoreType.DMA,
            pltpu.SemaphoreType.DMA,
        ],
    )
    return deg, edge


def _dis_block(cnt_ref, pid):
    cnt = cnt_ref[0] + cnt_ref[1]
    row = lax.broadcasted_iota(jnp.int32, (BLK, 1), 0) + pid * BLK
    return jnp.where(row < N, lax.rsqrt(cnt + 1.0), 0.0)


def _head_body(cnt_ref, x_ref, w_ref, out_ref):
    dis = _dis_block(cnt_ref, pl.program_id(0))
    out_ref[...] = dis * jnp.dot(x_ref[...], w_ref[...],
                                 preferred_element_type=jnp.float32)


def _layer_body(cnt_ref, acc_ref, hs_ref, b_ref, w_ref, out_ref):
    dis = _dis_block(cnt_ref, pl.program_id(0))
    t = dis * (acc_ref[0] + acc_ref[1] + hs_ref[...]) + b_ref[...]
    a = jnp.maximum(t, 0.0)
    out_ref[...] = dis * jnp.dot(a, w_ref[...],
                                 preferred_element_type=jnp.float32)


def _tail_body(cnt_ref, acc_ref, hs_ref, b_ref, out_ref):
    dis = _dis_block(cnt_ref, pl.program_id(0))
    out_ref[...] = dis * (acc_ref[0] + acc_ref[1] + hs_ref[...]) + b_ref[...]


_CNT_SPEC = pl.BlockSpec((2, BLK, 1), lambda i: (0, i, 0))
_ROW_SPEC = pl.BlockSpec((BLK, D), lambda i: (i, 0))
_ACC_SPEC = pl.BlockSpec((2, BLK, D), lambda i: (0, i, 0))
_W_SPEC = pl.BlockSpec((D, D), lambda i: (0, 0))
_B_SPEC = pl.BlockSpec((1, D), lambda i: (0, 0))
_GRID = NP // BLK
_OUT_SHAPE = jax.ShapeDtypeStruct((NP, D), jnp.float32)

_head = pl.pallas_call(
    _head_body, grid=(_GRID,),
    in_specs=[_CNT_SPEC, _ROW_SPEC, _W_SPEC],
    out_specs=_ROW_SPEC, out_shape=_OUT_SHAPE)

_layer = pl.pallas_call(
    _layer_body, grid=(_GRID,),
    in_specs=[_CNT_SPEC, _ACC_SPEC, _ROW_SPEC, _B_SPEC, _W_SPEC],
    out_specs=_ROW_SPEC, out_shape=_OUT_SHAPE)

_tail = pl.pallas_call(
    _tail_body, grid=(_GRID,),
    in_specs=[_CNT_SPEC, _ACC_SPEC, _ROW_SPEC, _B_SPEC],
    out_specs=_ROW_SPEC, out_shape=_OUT_SHAPE)


def kernel(x, edge_index, W1, b1, W2, b2, W3, b3):
    src = edge_index[0].astype(jnp.int32)
    dst = edge_index[1].astype(jnp.int32)
    pad = jnp.full((E_PAD - E,), N, jnp.int32)
    src_p = jnp.concatenate([src, pad])
    dst_p = jnp.concatenate([dst, pad])
    xp = jnp.pad(x, ((0, NP - N), (0, 0)))

    _deg, _edge_pass = _sc_kernels()
    cnt = _deg(dst_p)[:, :, None]

    hs1 = _head(cnt, xp, W1)
    acc1 = _edge_pass(hs1, src_p, dst_p)
    hs2 = _layer(cnt, acc1, hs1, b1[None, :], W2)
    acc2 = _edge_pass(hs2, src_p, dst_p)
    hs3 = _layer(cnt, acc2, hs2, b2[None, :], W3)
    acc3 = _edge_pass(hs3, src_p, dst_p)
    out = _tail(cnt, acc3, hs3, b3[None, :])
    return out[:N]


# speedup vs baseline: 2.0513x; 1.0949x over previous
import functools

import jax
import jax.numpy as jnp
from jax import lax
from jax.experimental import pallas as pl
from jax.experimental.pallas import tpu as pltpu
from jax.experimental.pallas import tpu_sc as plsc

N = 10000
D = 128
NP = 10240
E = 320000
NW = 32
CH = 128
NCH = 79
NCH0 = 100
NCH1 = 2 * NCH - NCH0
PER_W = CH * NCH
E_PAD = NW * PER_W
RPT = NP // 16
BLK = 1024

def _deg_body(dst_hbm, out_hbm, dst_v, ones_v, zeros_v, acc_sh):
    c = lax.axis_index("c")
    s = lax.axis_index("s")
    wid = c * 16 + s
    for k in range(CH // 16):
        ones_v[pl.ds(k * 16, 16)] = jnp.ones((16,), jnp.float32)
    for k in range(RPT // 16):
        zeros_v[pl.ds(k * 16, 16)] = jnp.zeros((16,), jnp.float32)
    pltpu.sync_copy(zeros_v, acc_sh.at[pl.ds(s * RPT, RPT)])
    plsc.subcore_barrier()

    def body(j, carry):
        off = wid * PER_W + j * CH
        pltpu.sync_copy(dst_hbm.at[pl.ds(off, CH)], dst_v)
        pltpu.sync_copy(ones_v, acc_sh.at[dst_v], add=True)
        return carry

    lax.fori_loop(0, NCH, body, 0)
    plsc.subcore_barrier()
    pltpu.sync_copy(acc_sh.at[pl.ds(s * RPT, RPT)],
                    out_hbm.at[c, pl.ds(s * RPT, RPT)])


def _edge_body(hs_hbm, src_hbm, dst_hbm, out_hbm,
               src_a, dst_a, src_b, dst_b, rows_v, zero_v, acc_sh,
               sem, sem_ia, sem_ib):
    c = lax.axis_index("c")
    s = lax.axis_index("s")
    nch = jnp.where(c == 0, NCH0, NCH1)
    base = jnp.where(c == 0, s * NCH0, 16 * NCH0 + s * NCH1) * CH
    z16 = jnp.zeros((16,), jnp.float32)
    for r in range(16):
        for k in range(D // 16):
            zero_v[r, pl.ds(k * 16, 16)] = z16

    def zbody(i, carry):
        pltpu.sync_copy(zero_v, acc_sh.at[pl.ds(s * RPT + i * 16, 16)])
        return carry

    lax.fori_loop(0, RPT // 16, zbody, 0)

    def load_idx(j, sv, dv, sem_i):
        pltpu.async_copy(src_hbm.at[pl.ds(base + j * CH, CH)], sv, sem_i)
        pltpu.async_copy(dst_hbm.at[pl.ds(base + j * CH, CH)], dv, sem_i)

    def wait_idx(sv, dv, sem_i):
        pltpu.make_async_copy(src_hbm.at[pl.ds(base, CH)], sv, sem_i).wait()
        pltpu.make_async_copy(dst_hbm.at[pl.ds(base, CH)], dv, sem_i).wait()

    def do_chunk(sv, dv):
        pltpu.async_copy(hs_hbm.at[sv], rows_v, sem).wait()
        pltpu.sync_copy(rows_v, acc_sh.at[dv], add=True)

    load_idx(0, src_a, dst_a, sem_ia)
    load_idx(1, src_b, dst_b, sem_ib)
    plsc.subcore_barrier()

    def body(k, carry):
        j0 = 2 * k
        wait_idx(src_a, dst_a, sem_ia)
        do_chunk(src_a, dst_a)

        @pl.when(j0 + 2 < nch)
        def _():
            pltpu.async_copy(src_hbm.at[pl.ds(base + (j0 + 2) * CH, CH)],
                             src_a, sem_ia)
            pltpu.async_copy(dst_hbm.at[pl.ds(base + (j0 + 2) * CH, CH)],
                             dst_a, sem_ia)

        wait_idx(src_b, dst_b, sem_ib)
        do_chunk(src_b, dst_b)

        @pl.when(j0 + 3 < nch)
        def _():
            pltpu.async_copy(src_hbm.at[pl.ds(base + (j0 + 3) * CH, CH)],
                             src_b, sem_ib)
            pltpu.async_copy(dst_hbm.at[pl.ds(base + (j0 + 3) * CH, CH)],
                             dst_b, sem_ib)
        return carry

    lax.fori_loop(0, nch // 2, body, 0)
    plsc.subcore_barrier()
    pltpu.sync_copy(acc_sh.at[pl.ds(s * RPT, RPT)],
                    out_hbm.at[c, pl.ds(s * RPT, RPT)])


@functools.cache
def _sc_kernels():
    mesh = plsc.VectorSubcoreMesh(core_axis_name="c", subcore_axis_name="s",
                                  num_cores=2, num_subcores=16)
    deg = pl.kernel(
        _deg_body,
        out_type=jax.ShapeDtypeStruct((2, NP), jnp.float32),
        mesh=mesh,
        scratch_types=[
            pltpu.VMEM((CH,), jnp.int32),
            pltpu.VMEM((CH,), jnp.float32),
            pltpu.VMEM((RPT,), jnp.float32),
            pltpu.VMEM_SHARED((NP,), jnp.float32),
        ],
    )
    edge = pl.kernel(
        _edge_body,
        out_type=jax.ShapeDtypeStruct((2, NP, D), jnp.float32),
        mesh=mesh,
        scratch_types=[
            pltpu.VMEM((CH,), jnp.int32),
            pltpu.VMEM((CH,), jnp.int32),
            pltpu.VMEM((CH,), jnp.int32),
            pltpu.VMEM((CH,), jnp.int32),
            pltpu.VMEM((CH, D), jnp.float32),
            pltpu.VMEM((16, D), jnp.float32),
            pltpu.VMEM_SHARED((NP, D), jnp.float32),
            pltpu.SemaphoreType.DMA,
            pltpu.SemaphoreType.DMA,
            pltpu.SemaphoreType.DMA,
        ],
    )
    return deg, edge


def _dis_block(cnt_ref, pid):
    cnt = cnt_ref[0] + cnt_ref[1]
    row = lax.broadcasted_iota(jnp.int32, (BLK, 1), 0) + pid * BLK
    return jnp.where(row < N, lax.rsqrt(cnt + 1.0), 0.0)


def _head_body(cnt_ref, x_ref, w_ref, out_ref):
    dis = _dis_block(cnt_ref, pl.program_id(0))
    out_ref[...] = dis * jnp.dot(x_ref[...], w_ref[...],
                                 preferred_element_type=jnp.float32)


def _layer_body(cnt_ref, acc_ref, hs_ref, b_ref, w_ref, out_ref):
    dis = _dis_block(cnt_ref, pl.program_id(0))
    t = dis * (acc_ref[0] + acc_ref[1] + hs_ref[...]) + b_ref[...]
    a = jnp.maximum(t, 0.0)
    out_ref[...] = dis * jnp.dot(a, w_ref[...],
                                 preferred_element_type=jnp.float32)


def _tail_body(cnt_ref, acc_ref, hs_ref, b_ref, out_ref):
    dis = _dis_block(cnt_ref, pl.program_id(0))
    out_ref[...] = dis * (acc_ref[0] + acc_ref[1] + hs_ref[...]) + b_ref[...]


_CNT_SPEC = pl.BlockSpec((2, BLK, 1), lambda i: (0, i, 0))
_ROW_SPEC = pl.BlockSpec((BLK, D), lambda i: (i, 0))
_ACC_SPEC = pl.BlockSpec((2, BLK, D), lambda i: (0, i, 0))
_W_SPEC = pl.BlockSpec((D, D), lambda i: (0, 0))
_B_SPEC = pl.BlockSpec((1, D), lambda i: (0, 0))
_GRID = NP // BLK
_OUT_SHAPE = jax.ShapeDtypeStruct((NP, D), jnp.float32)

_head = pl.pallas_call(
    _head_body, grid=(_GRID,),
    in_specs=[_CNT_SPEC, _ROW_SPEC, _W_SPEC],
    out_specs=_ROW_SPEC, out_shape=_OUT_SHAPE)

_layer = pl.pallas_call(
    _layer_body, grid=(_GRID,),
    in_specs=[_CNT_SPEC, _ACC_SPEC, _ROW_SPEC, _B_SPEC, _W_SPEC],
    out_specs=_ROW_SPEC, out_shape=_OUT_SHAPE)

_tail = pl.pallas_call(
    _tail_body, grid=(_GRID,),
    in_specs=[_CNT_SPEC, _ACC_SPEC, _ROW_SPEC, _B_SPEC],
    out_specs=_ROW_SPEC, out_shape=_OUT_SHAPE)


def kernel(x, edge_index, W1, b1, W2, b2, W3, b3):
    src = edge_index[0].astype(jnp.int32)
    dst = edge_index[1].astype(jnp.int32)
    pad = jnp.full((E_PAD - E,), N, jnp.int32)
    src_p = jnp.concatenate([src, pad])
    dst_p = jnp.concatenate([dst, pad])
    xp = jnp.pad(x, ((0, NP - N), (0, 0)))

    _deg, _edge_pass = _sc_kernels()
    cnt = _deg(dst_p)[:, :, None]

    hs1 = _head(cnt, xp, W1)
    acc1 = _edge_pass(hs1, src_p, dst_p)
    hs2 = _layer(cnt, acc1, hs1, b1[None, :], W2)
    acc2 = _edge_pass(hs2, src_p, dst_p)
    hs3 = _layer(cnt, acc2, hs2, b2[None, :], W3)
    acc3 = _edge_pass(hs3, src_p, dst_p)
    out = _tail(cnt, acc3, hs3, b3[None, :])
    return out[:N]


# speedup vs baseline: 2.1227x; 1.0348x over previous
import functools

import jax
import jax.numpy as jnp
from jax import lax
from jax.experimental import pallas as pl
from jax.experimental.pallas import tpu as pltpu
from jax.experimental.pallas import tpu_sc as plsc

N = 10000
D = 128
NP = 10240
E = 320000
NW = 32
CH = 128
NCH = 79
NCH0 = 120
NCH1 = 2 * NCH - NCH0
PER_W = CH * NCH
E_PAD = NW * PER_W
RPT = NP // 16
BLK = 1024

def _deg_body(dst_hbm, out_hbm, dst_v, ones_v, zeros_v, acc_sh):
    c = lax.axis_index("c")
    s = lax.axis_index("s")
    wid = c * 16 + s
    for k in range(CH // 16):
        ones_v[pl.ds(k * 16, 16)] = jnp.ones((16,), jnp.float32)
    for k in range(RPT // 16):
        zeros_v[pl.ds(k * 16, 16)] = jnp.zeros((16,), jnp.float32)
    pltpu.sync_copy(zeros_v, acc_sh.at[pl.ds(s * RPT, RPT)])
    plsc.subcore_barrier()

    def body(j, carry):
        off = wid * PER_W + j * CH
        pltpu.sync_copy(dst_hbm.at[pl.ds(off, CH)], dst_v)
        pltpu.sync_copy(ones_v, acc_sh.at[dst_v], add=True)
        return carry

    lax.fori_loop(0, NCH, body, 0)
    plsc.subcore_barrier()
    pltpu.sync_copy(acc_sh.at[pl.ds(s * RPT, RPT)],
                    out_hbm.at[c, pl.ds(s * RPT, RPT)])


def _edge_body(hs_hbm, src_hbm, dst_hbm, out_hbm,
               src_a, dst_a, src_b, dst_b, rows_v, zero_v, acc_sh,
               sem, sem_ia, sem_ib):
    c = lax.axis_index("c")
    s = lax.axis_index("s")
    nch = jnp.where(c == 0, NCH0, NCH1)
    base = jnp.where(c == 0, s * NCH0, 16 * NCH0 + s * NCH1) * CH
    z16 = jnp.zeros((16,), jnp.float32)
    for r in range(16):
        for k in range(D // 16):
            zero_v[r, pl.ds(k * 16, 16)] = z16

    def zbody(i, carry):
        pltpu.sync_copy(zero_v, acc_sh.at[pl.ds(s * RPT + i * 16, 16)])
        return carry

    lax.fori_loop(0, RPT // 16, zbody, 0)

    def load_idx(j, sv, dv, sem_i):
        pltpu.async_copy(src_hbm.at[pl.ds(base + j * CH, CH)], sv, sem_i)
        pltpu.async_copy(dst_hbm.at[pl.ds(base + j * CH, CH)], dv, sem_i)

    def wait_idx(sv, dv, sem_i):
        pltpu.make_async_copy(src_hbm.at[pl.ds(base, CH)], sv, sem_i).wait()
        pltpu.make_async_copy(dst_hbm.at[pl.ds(base, CH)], dv, sem_i).wait()

    def do_chunk(sv, dv):
        pltpu.async_copy(hs_hbm.at[sv], rows_v, sem).wait()
        pltpu.sync_copy(rows_v, acc_sh.at[dv], add=True)

    load_idx(0, src_a, dst_a, sem_ia)
    load_idx(1, src_b, dst_b, sem_ib)
    plsc.subcore_barrier()

    def body(k, carry):
        j0 = 2 * k
        wait_idx(src_a, dst_a, sem_ia)
        do_chunk(src_a, dst_a)

        @pl.when(j0 + 2 < nch)
        def _():
            pltpu.async_copy(src_hbm.at[pl.ds(base + (j0 + 2) * CH, CH)],
                             src_a, sem_ia)
            pltpu.async_copy(dst_hbm.at[pl.ds(base + (j0 + 2) * CH, CH)],
                             dst_a, sem_ia)

        wait_idx(src_b, dst_b, sem_ib)
        do_chunk(src_b, dst_b)

        @pl.when(j0 + 3 < nch)
        def _():
            pltpu.async_copy(src_hbm.at[pl.ds(base + (j0 + 3) * CH, CH)],
                             src_b, sem_ib)
            pltpu.async_copy(dst_hbm.at[pl.ds(base + (j0 + 3) * CH, CH)],
                             dst_b, sem_ib)
        return carry

    lax.fori_loop(0, nch // 2, body, 0)
    plsc.subcore_barrier()
    pltpu.sync_copy(acc_sh.at[pl.ds(s * RPT, RPT)],
                    out_hbm.at[c, pl.ds(s * RPT, RPT)])


@functools.cache
def _sc_kernels():
    mesh = plsc.VectorSubcoreMesh(core_axis_name="c", subcore_axis_name="s",
                                  num_cores=2, num_subcores=16)
    deg = pl.kernel(
        _deg_body,
        out_type=jax.ShapeDtypeStruct((2, NP), jnp.float32),
        mesh=mesh,
        scratch_types=[
            pltpu.VMEM((CH,), jnp.int32),
            pltpu.VMEM((CH,), jnp.float32),
            pltpu.VMEM((RPT,), jnp.float32),
            pltpu.VMEM_SHARED((NP,), jnp.float32),
        ],
    )
    edge = pl.kernel(
        _edge_body,
        out_type=jax.ShapeDtypeStruct((2, NP, D), jnp.float32),
        mesh=mesh,
        scratch_types=[
            pltpu.VMEM((CH,), jnp.int32),
            pltpu.VMEM((CH,), jnp.int32),
            pltpu.VMEM((CH,), jnp.int32),
            pltpu.VMEM((CH,), jnp.int32),
            pltpu.VMEM((CH, D), jnp.float32),
            pltpu.VMEM((16, D), jnp.float32),
            pltpu.VMEM_SHARED((NP, D), jnp.float32),
            pltpu.SemaphoreType.DMA,
            pltpu.SemaphoreType.DMA,
            pltpu.SemaphoreType.DMA,
        ],
    )
    return deg, edge


def _dis_block(cnt_ref, pid):
    cnt = cnt_ref[0] + cnt_ref[1]
    row = lax.broadcasted_iota(jnp.int32, (BLK, 1), 0) + pid * BLK
    return jnp.where(row < N, lax.rsqrt(cnt + 1.0), 0.0)


def _head_body(cnt_ref, x_ref, w_ref, out_ref):
    dis = _dis_block(cnt_ref, pl.program_id(0))
    out_ref[...] = dis * jnp.dot(x_ref[...], w_ref[...],
                                 preferred_element_type=jnp.float32)


def _layer_body(cnt_ref, acc_ref, hs_ref, b_ref, w_ref, out_ref):
    dis = _dis_block(cnt_ref, pl.program_id(0))
    t = dis * (acc_ref[0] + acc_ref[1] + hs_ref[...]) + b_ref[...]
    a = jnp.maximum(t, 0.0)
    out_ref[...] = dis * jnp.dot(a, w_ref[...],
                                 preferred_element_type=jnp.float32)


def _tail_body(cnt_ref, acc_ref, hs_ref, b_ref, out_ref):
    dis = _dis_block(cnt_ref, pl.program_id(0))
    out_ref[...] = dis * (acc_ref[0] + acc_ref[1] + hs_ref[...]) + b_ref[...]


_CNT_SPEC = pl.BlockSpec((2, BLK, 1), lambda i: (0, i, 0))
_ROW_SPEC = pl.BlockSpec((BLK, D), lambda i: (i, 0))
_ACC_SPEC = pl.BlockSpec((2, BLK, D), lambda i: (0, i, 0))
_W_SPEC = pl.BlockSpec((D, D), lambda i: (0, 0))
_B_SPEC = pl.BlockSpec((1, D), lambda i: (0, 0))
_GRID = NP // BLK
_OUT_SHAPE = jax.ShapeDtypeStruct((NP, D), jnp.float32)

_head = pl.pallas_call(
    _head_body, grid=(_GRID,),
    in_specs=[_CNT_SPEC, _ROW_SPEC, _W_SPEC],
    out_specs=_ROW_SPEC, out_shape=_OUT_SHAPE)

_layer = pl.pallas_call(
    _layer_body, grid=(_GRID,),
    in_specs=[_CNT_SPEC, _ACC_SPEC, _ROW_SPEC, _B_SPEC, _W_SPEC],
    out_specs=_ROW_SPEC, out_shape=_OUT_SHAPE)

_tail = pl.pallas_call(
    _tail_body, grid=(_GRID,),
    in_specs=[_CNT_SPEC, _ACC_SPEC, _ROW_SPEC, _B_SPEC],
    out_specs=_ROW_SPEC, out_shape=_OUT_SHAPE)


def kernel(x, edge_index, W1, b1, W2, b2, W3, b3):
    src = edge_index[0].astype(jnp.int32)
    dst = edge_index[1].astype(jnp.int32)
    pad = jnp.full((E_PAD - E,), N, jnp.int32)
    src_p = jnp.concatenate([src, pad])
    dst_p = jnp.concatenate([dst, pad])
    xp = jnp.pad(x, ((0, NP - N), (0, 0)))

    _deg, _edge_pass = _sc_kernels()
    cnt = _deg(dst_p)[:, :, None]

    hs1 = _head(cnt, xp, W1)
    acc1 = _edge_pass(hs1, src_p, dst_p)
    hs2 = _layer(cnt, acc1, hs1, b1[None, :], W2)
    acc2 = _edge_pass(hs2, src_p, dst_p)
    hs3 = _layer(cnt, acc2, hs2, b2[None, :], W3)
    acc3 = _edge_pass(hs3, src_p, dst_p)
    out = _tail(cnt, acc3, hs3, b3[None, :])
    return out[:N]
